# baseline (device time: 28134 ns/iter reference)
import jax
import jax.numpy as jnp
from jax import lax
from jax.experimental import pallas as pl
from jax.experimental.pallas import tpu as pltpu

N_DEV = 32
SUB, BLK = 128, 16


def kernel(x, gamma):
    m, n_per = x.shape
    n_global = n_per * N_DEV
    eps = 1e-5

    def body(x_ref, g_ref, out_ref, part_ref, comm_ref, send_sems, recv_sems):
        my_i = lax.axis_index("i")

        xx = x_ref[...]
        part = jnp.sum(xx * xx, axis=1)
        part_ref[...] = part.reshape(BLK, SUB)

        rdmas = []
        for d in range(1, N_DEV):
            rdma = pltpu.make_async_remote_copy(
                src_ref=part_ref,
                dst_ref=comm_ref.at[d - 1],
                send_sem=send_sems.at[d - 1],
                recv_sem=recv_sems.at[d - 1],
                device_id=((my_i + d) % N_DEV,),
                device_id_type=pl.DeviceIdType.MESH,
            )
            rdma.start()
            rdmas.append(rdma)

        comm_ref[N_DEV - 1, :, :] = part_ref[...]

        out_ref[...] = xx * g_ref[...]

        for rdma in rdmas:
            rdma.wait()

        total = jnp.sum(comm_ref[...], axis=0)
        inv = lax.rsqrt(total / n_global + eps)
        eye = (
            lax.broadcasted_iota(jnp.int32, (BLK, BLK), 0)
            == lax.broadcasted_iota(jnp.int32, (BLK, BLK), 1)
        ).astype(jnp.float32)
        tt = lax.dot_general(
            inv,
            eye,
            (((0,), (0,)), ((), ())),
            preferred_element_type=jnp.float32,
        )
        for i in range(BLK):
            sl = pl.ds(i * SUB, SUB)
            out_ref[sl, :] = out_ref[sl, :] * tt[:, i : i + 1]

    return pl.pallas_call(
        body,
        out_shape=jax.ShapeDtypeStruct((m, n_per), jnp.float32),
        in_specs=[
            pl.BlockSpec(memory_space=pltpu.VMEM),
            pl.BlockSpec(memory_space=pltpu.VMEM),
        ],
        out_specs=pl.BlockSpec(memory_space=pltpu.VMEM),
        scratch_shapes=[
            pltpu.VMEM((BLK, SUB), jnp.float32),
            pltpu.VMEM((N_DEV, BLK, SUB), jnp.float32),
            pltpu.SemaphoreType.DMA((N_DEV - 1,)),
            pltpu.SemaphoreType.DMA((N_DEV - 1,)),
        ],
    )(x, gamma.reshape(1, n_per))


# device time: 22091 ns/iter; 1.2736x vs baseline; 1.2736x over previous
import jax
import jax.numpy as jnp
from jax import lax
from jax.experimental import pallas as pl
from jax.experimental.pallas import tpu as pltpu

N_DEV = 32
SUB, BLK = 128, 16


def kernel(x, gamma):
    m, n_per = x.shape
    n_global = n_per * N_DEV
    eps = 1e-5

    def body(x_ref, g_ref, out_ref, part_ref, comm_ref, send_sems, recv_sems):
        my_i = lax.axis_index("i")

        xx = x_ref[...]
        part = jnp.sum(xx * xx, axis=1)
        part_ref[...] = part.reshape(BLK, SUB)

        barrier_sem = pltpu.get_barrier_semaphore()
        for d in range(1, N_DEV):
            pl.semaphore_signal(
                barrier_sem,
                inc=1,
                device_id=((my_i + d) % N_DEV,),
                device_id_type=pl.DeviceIdType.MESH,
            )

        comm_ref[N_DEV - 1, :, :] = part_ref[...]
        out_ref[...] = xx * g_ref[...]

        pl.semaphore_wait(barrier_sem, N_DEV - 1)

        rdmas = []
        for d in range(1, N_DEV):
            rdma = pltpu.make_async_remote_copy(
                src_ref=part_ref,
                dst_ref=comm_ref.at[d - 1],
                send_sem=send_sems.at[d - 1],
                recv_sem=recv_sems.at[d - 1],
                device_id=((my_i + d) % N_DEV,),
                device_id_type=pl.DeviceIdType.MESH,
            )
            rdma.start()
            rdmas.append(rdma)

        for rdma in rdmas:
            rdma.wait()

        total = jnp.sum(comm_ref[...], axis=0)
        inv = lax.rsqrt(total / n_global + eps)
        eye = (
            lax.broadcasted_iota(jnp.int32, (BLK, BLK), 0)
            == lax.broadcasted_iota(jnp.int32, (BLK, BLK), 1)
        ).astype(jnp.float32)
        tt = lax.dot_general(
            inv,
            eye,
            (((0,), (0,)), ((), ())),
            preferred_element_type=jnp.float32,
        )
        for i in range(BLK):
            sl = pl.ds(i * SUB, SUB)
            out_ref[sl, :] = out_ref[sl, :] * tt[:, i : i + 1]

    return pl.pallas_call(
        body,
        out_shape=jax.ShapeDtypeStruct((m, n_per), jnp.float32),
        in_specs=[
            pl.BlockSpec(memory_space=pltpu.VMEM),
            pl.BlockSpec(memory_space=pltpu.VMEM),
        ],
        out_specs=pl.BlockSpec(memory_space=pltpu.VMEM),
        scratch_shapes=[
            pltpu.VMEM((BLK, SUB), jnp.float32),
            pltpu.VMEM((N_DEV, BLK, SUB), jnp.float32),
            pltpu.SemaphoreType.DMA((N_DEV - 1,)),
            pltpu.SemaphoreType.DMA((N_DEV - 1,)),
        ],
        compiler_params=pltpu.CompilerParams(collective_id=0),
    )(x, gamma.reshape(1, n_per))
